# Initial kernel scaffold; baseline (speedup 1.0000x reference)
import jax
import jax.numpy as jnp
from jax.experimental import pallas as pl


def kernel(x, edge_index, gamma, beta, eW1, eb1, eW2, eb2, eW3, eb3, dW1, db1, dW2, db2, dW3, db3):
    raise NotImplementedError("write your pallas kernel here")


# speedup vs baseline: 1.0237x; 1.0237x over previous
import functools

import jax
import jax.numpy as jnp
from jax.experimental import pallas as pl

_N = 100000
_E = 1600000


def _mlp_body(xi_ref, xj_ref, wd_ref, ws_ref, b1_ref, w2_ref, b2_ref,
              w3_ref, b3_ref, o_ref, *, relu_out):
    xi = xi_ref[...]
    xj = xj_ref[...]
    u = (jnp.dot(xi, wd_ref[...], preferred_element_type=jnp.float32)
         + jnp.dot(xj, ws_ref[...], preferred_element_type=jnp.float32)
         + b1_ref[...])
    h = jnp.maximum(u, 0.0)
    h = jnp.maximum(
        jnp.dot(h, w2_ref[...], preferred_element_type=jnp.float32)
        + b2_ref[...], 0.0)
    o = jnp.dot(h, w3_ref[...], preferred_element_type=jnp.float32) + b3_ref[...]
    if relu_out:
        o = jnp.maximum(o, 0.0)
    o_ref[...] = o


def _pick_block(e):
    for b in (8000, 6400, 5000, 4000, 3200, 2000, 1600, 1000, 800, 500, 400,
              200, 100, 50, 25, 10, 5, 1):
        if e % b == 0:
            return b
    return 1


def _edge_mlp(xi, xj, wd, ws, b1, w2, b2, w3, b3, relu_out):
    e, din = xi.shape
    dout = w3.shape[1]
    blk = _pick_block(e)
    grid = (e // blk,)
    full = lambda r, c: pl.BlockSpec((r, c), lambda i: (0, 0))
    return pl.pallas_call(
        functools.partial(_mlp_body, relu_out=relu_out),
        grid=grid,
        in_specs=[
            pl.BlockSpec((blk, din), lambda i: (i, 0)),
            pl.BlockSpec((blk, din), lambda i: (i, 0)),
            full(din, 32), full(din, 32), full(1, 32),
            full(32, 32), full(1, 32),
            full(32, dout), full(1, dout),
        ],
        out_specs=pl.BlockSpec((blk, dout), lambda i: (i, 0)),
        out_shape=jax.ShapeDtypeStruct((e, dout), jnp.float32),
    )(xi, xj, wd, ws, b1.reshape(1, -1), w2, b2.reshape(1, -1),
      w3, b3.reshape(1, -1))


def kernel(x, edge_index, gamma, beta, eW1, eb1, eW2, eb2, eW3, eb3,
           dW1, db1, dW2, db2, dW3, db3):
    src = edge_index[0]
    dst = edge_index[1]
    n, d = x.shape

    mean = jnp.mean(x, axis=0)
    var = jnp.var(x, axis=0)
    a = gamma * jax.lax.rsqrt(var + 1e-5)
    c = beta - mean * a

    w1t = eW1.T
    w1s = w1t[d:]
    w1d = w1t[:d] - w1s
    w1d_f = a[:, None] * w1d
    w1s_f = a[:, None] * w1s
    b1_f = eb1 + c @ w1t[:d]

    xi = jnp.take(x, dst, axis=0)
    xj = jnp.take(x, src, axis=0)
    m1 = _edge_mlp(xi, xj, w1d_f, w1s_f, b1_f, eW2.T, eb2, eW3.T, eb3, True)

    ones = jnp.ones((m1.shape[0],), jnp.float32)
    cnt = jax.ops.segment_sum(ones, dst, num_segments=n)
    s1 = jax.ops.segment_sum(m1, dst, num_segments=n)
    inv = 1.0 / jnp.clip(cnt, 1.0)
    h1 = s1 * inv[:, None]

    w1t2 = dW1.T
    hid = h1.shape[1]
    w2s = w1t2[hid:]
    w2d = w1t2[:hid] - w2s

    hi = jnp.take(h1, dst, axis=0)
    hj = jnp.take(h1, src, axis=0)
    m2 = _edge_mlp(hi, hj, w2d, w2s, db1, dW2.T, db2, dW3.T, db3, False)

    s2 = jax.ops.segment_sum(m2, dst, num_segments=n)
    return s2 * inv[:, None]


# speedup vs baseline: 4.0309x; 3.9376x over previous
import functools

import jax
import jax.numpy as jnp
from jax import lax
from jax.experimental import pallas as pl
from jax.experimental.pallas import tpu as pltpu

_N = 100000
_E = 1600000
_NC = 2
_NS = 16
_NW = _NC * _NS
_K = 128
_CH = 8
_DP = 16
_N_PAD = 100352
_E_PAD = 1605632
_GR = 2 * _E_PAD // _K
_GRT = _GR // _NW
_GSTEP = _GRT // _CH
_SR = _E_PAD // _K
_SRT = _SR // _NW
_SSTEP = _SRT // _CH
_RPT = _N_PAD // _NS


def _sc_mesh():
    from jax.experimental.pallas import tpu_sc as plsc
    return plsc.VectorSubcoreMesh(core_axis_name="c", subcore_axis_name="s")


def _gather_body(tab_hbm, idx_hbm, out_hbm, idx_v, rows_v, sem):
    c = lax.axis_index("c")
    s = lax.axis_index("s")
    wid = s * _NC + c

    @pl.loop(0, _GSTEP)
    def _step(t):
        r0 = wid * _GRT + t * _CH
        pltpu.sync_copy(idx_hbm.at[pl.ds(r0, _CH)], idx_v)
        handles = [
            pltpu.async_copy(tab_hbm.at[idx_v.at[j]],
                             rows_v.at[pl.ds(j * _K, _K)], sem)
            for j in range(_CH)
        ]
        for h in handles:
            h.wait()
        pltpu.sync_copy(rows_v, out_hbm.at[pl.ds(r0 * _K, _CH * _K)])


def _sc_gather(tab, gidx):
    f = pl.kernel(
        _gather_body,
        out_type=jax.ShapeDtypeStruct((2 * _E_PAD, _DP), jnp.float32),
        mesh=_sc_mesh(),
        compiler_params=pltpu.CompilerParams(use_tc_tiling_on_sc=False),
        scratch_types=[
            pltpu.VMEM((_CH, _K), jnp.int32),
            pltpu.VMEM((_CH * _K, _DP), jnp.float32),
            pltpu.SemaphoreType.DMA,
        ],
    )
    return f(tab, gidx)


def _scatter_body(vals_hbm, idx_hbm, zeros_hbm, part_hbm,
                  acc_sh, idx_v, vals_v, sem):
    from jax.experimental.pallas import tpu_sc as plsc
    c = lax.axis_index("c")
    s = lax.axis_index("s")
    wid = s * _NC + c
    pltpu.sync_copy(zeros_hbm.at[pl.ds(s * _RPT, _RPT)],
                    acc_sh.at[pl.ds(s * _RPT, _RPT)])
    plsc.subcore_barrier()

    @pl.loop(0, _SSTEP)
    def _step(t):
        r0 = wid * _SRT + t * _CH
        pltpu.sync_copy(idx_hbm.at[pl.ds(r0, _CH)], idx_v)
        pltpu.sync_copy(vals_hbm.at[pl.ds(r0 * _K, _CH * _K)], vals_v)
        for j in range(_CH):
            pltpu.sync_copy(vals_v.at[pl.ds(j * _K, _K)],
                            acc_sh.at[idx_v.at[j]], add=True)

    plsc.subcore_barrier()
    pltpu.sync_copy(acc_sh.at[pl.ds(s * _RPT, _RPT)],
                    part_hbm.at[c, pl.ds(s * _RPT, _RPT)])


def _sc_scatter(vals, sidx, zeros16):
    f = pl.kernel(
        _scatter_body,
        out_type=jax.ShapeDtypeStruct((_NC, _N_PAD, _DP), jnp.float32),
        mesh=_sc_mesh(),
        compiler_params=pltpu.CompilerParams(use_tc_tiling_on_sc=False),
        scratch_types=[
            pltpu.VMEM_SHARED((_N_PAD, _DP), jnp.float32),
            pltpu.VMEM((_CH, _K), jnp.int32),
            pltpu.VMEM((_CH * _K, _DP), jnp.float32),
            pltpu.SemaphoreType.DMA,
        ],
    )
    return f(vals, sidx, zeros16)


def _mlp_body(xc_ref, w1_ref, b1_ref, w2_ref, b2_ref, w3_ref, b3_ref,
              o_ref, *, relu_out):
    u = (jnp.dot(xc_ref[...], w1_ref[...], preferred_element_type=jnp.float32)
         + b1_ref[...])
    h = jnp.maximum(u, 0.0)
    h = jnp.maximum(
        jnp.dot(h, w2_ref[...], preferred_element_type=jnp.float32)
        + b2_ref[...], 0.0)
    o = jnp.dot(h, w3_ref[...], preferred_element_type=jnp.float32) + b3_ref[...]
    if relu_out:
        o = jnp.maximum(o, 0.0)
    o_ref[...] = o


_MLP_BLK = 4096


def _edge_mlp(xc, w1, b1, w2, b2, w3, b3, relu_out):
    full = lambda r, c: pl.BlockSpec((r, c), lambda i: (0, 0))
    return pl.pallas_call(
        functools.partial(_mlp_body, relu_out=relu_out),
        grid=(_E_PAD // _MLP_BLK,),
        in_specs=[
            pl.BlockSpec((_MLP_BLK, 32), lambda i: (i, 0)),
            full(32, 32), full(1, 32),
            full(32, 32), full(1, 32),
            full(32, _DP), full(1, _DP),
        ],
        out_specs=pl.BlockSpec((_MLP_BLK, _DP), lambda i: (i, 0)),
        out_shape=jax.ShapeDtypeStruct((_E_PAD, _DP), jnp.float32),
    )(xc, w1, b1.reshape(1, -1), w2, b2.reshape(1, -1),
      w3, b3.reshape(1, -1))


_NB = 6272


def _combine_body(p0_ref, p1_ref, h1_ref, inv_ref):
    s = p0_ref[...] + p1_ref[...]
    inv = 1.0 / jnp.maximum(s[:, 2:3], 1.0)
    cols = lax.broadcasted_iota(jnp.int32, (_NB, _DP), 1)
    h1_ref[...] = jnp.where(cols < 2, s * inv, 0.0)
    inv_ref[...] = inv


def _combine(p0, p1):
    return pl.pallas_call(
        _combine_body,
        grid=(_N_PAD // _NB,),
        in_specs=[pl.BlockSpec((_NB, _DP), lambda i: (i, 0)),
                  pl.BlockSpec((_NB, _DP), lambda i: (i, 0))],
        out_specs=[pl.BlockSpec((_NB, _DP), lambda i: (i, 0)),
                   pl.BlockSpec((_NB, 1), lambda i: (i, 0))],
        out_shape=[jax.ShapeDtypeStruct((_N_PAD, _DP), jnp.float32),
                   jax.ShapeDtypeStruct((_N_PAD, 1), jnp.float32)],
    )(p0, p1)


def _final_body(p0_ref, p1_ref, inv_ref, o_ref):
    s = p0_ref[...] + p1_ref[...]
    o_ref[...] = s[:, 0:4] * inv_ref[...]


def _final(p0, p1, inv):
    return pl.pallas_call(
        _final_body,
        grid=(_N_PAD // _NB,),
        in_specs=[pl.BlockSpec((_NB, _DP), lambda i: (i, 0)),
                  pl.BlockSpec((_NB, _DP), lambda i: (i, 0)),
                  pl.BlockSpec((_NB, 1), lambda i: (i, 0))],
        out_specs=pl.BlockSpec((_NB, 4), lambda i: (i, 0)),
        out_shape=jax.ShapeDtypeStruct((_N_PAD, 4), jnp.float32),
    )(p0, p1, inv)


def kernel(x, edge_index, gamma, beta, eW1, eb1, eW2, eb2, eW3, eb3,
           dW1, db1, dW2, db2, dW3, db3):
    n, d = x.shape

    npad = _E_PAD - _E
    dump = (_N + (jnp.arange(npad, dtype=jnp.int32) % 128))
    dst = jnp.concatenate([edge_index[1], dump])
    src = jnp.concatenate([edge_index[0], dump])
    gidx = jnp.stack([dst, src], axis=1).reshape(_GR, _K)
    sidx = dst.reshape(_SR, _K)
    x16 = jnp.zeros((_N_PAD, _DP), jnp.float32).at[:n, :d].set(x)
    zeros16 = jnp.zeros((_N_PAD, _DP), jnp.float32)

    mean = jnp.mean(x, axis=0)
    var = jnp.var(x, axis=0)
    a = gamma * lax.rsqrt(var + 1e-5)
    c = beta - mean * a
    w1t = eW1.T
    w1s = w1t[d:]
    w1d = w1t[:d] - w1s
    w1c = (jnp.zeros((32, 32), jnp.float32)
           .at[0:d].set(a[:, None] * w1d)
           .at[_DP:_DP + d].set(a[:, None] * w1s))
    b1c = eb1 + c @ (w1d + w1s)
    w3c = jnp.zeros((32, _DP), jnp.float32).at[:, 0:2].set(eW3.T)
    b3c = jnp.zeros((_DP,), jnp.float32).at[0:2].set(eb3).at[2].set(1.0)

    xc = _sc_gather(x16, gidx).reshape(_E_PAD, 2 * _DP)
    m1 = _edge_mlp(xc, w1c, b1c, eW2.T, eb2, w3c, b3c, True)
    p1 = _sc_scatter(m1, sidx, zeros16)
    h1_16, inv = _combine(p1[0], p1[1])

    hid = 2
    w1t2 = dW1.T
    w2s = w1t2[hid:]
    w2d = w1t2[:hid] - w2s
    w1c2 = (jnp.zeros((32, 32), jnp.float32)
            .at[0:hid].set(w2d)
            .at[_DP:_DP + hid].set(w2s))
    w3c2 = jnp.zeros((32, _DP), jnp.float32).at[:, 0:4].set(dW3.T)
    b3c2 = jnp.zeros((_DP,), jnp.float32).at[0:4].set(db3)

    hc = _sc_gather(h1_16, gidx).reshape(_E_PAD, 2 * _DP)
    m2 = _edge_mlp(hc, w1c2, db1, dW2.T, db2, w3c2, b3c2, False)
    p2 = _sc_scatter(m2, sidx, zeros16)
    out = _final(p2[0], p2[1], inv)
    return out[:n]


# speedup vs baseline: 4.4040x; 1.0925x over previous
import functools

import jax
import jax.numpy as jnp
from jax import lax
from jax.experimental import pallas as pl
from jax.experimental.pallas import tpu as pltpu

_N = 100000
_E = 1600000
_NC = 2
_NS = 16
_NW = _NC * _NS
_K = 128
_CH = 8
_DP = 16
_N_PAD = 100352
_E_PAD = 1605632
_SR = _E_PAD // _K
_SRT = _SR // _NW
_SSTEP = _SRT // _CH
_RPT = _N_PAD // _NS


def _sc_mesh():
    from jax.experimental.pallas import tpu_sc as plsc
    return plsc.VectorSubcoreMesh(core_axis_name="c", subcore_axis_name="s")


def _gather_body(tab_hbm, dsti_hbm, srci_hbm, xi_hbm, xj_hbm,
                 idx_v, rows_v, sem):
    c = lax.axis_index("c")
    s = lax.axis_index("s")
    wid = s * _NC + c
    for idx_hbm, out_hbm in ((dsti_hbm, xi_hbm), (srci_hbm, xj_hbm)):
        @pl.loop(0, _SSTEP)
        def _step(t, idx_hbm=idx_hbm, out_hbm=out_hbm):
            r0 = wid * _SRT + t * _CH
            pltpu.sync_copy(idx_hbm.at[pl.ds(r0, _CH)], idx_v)
            handles = [
                pltpu.async_copy(tab_hbm.at[idx_v.at[j]],
                                 rows_v.at[pl.ds(j * _K, _K)], sem)
                for j in range(_CH)
            ]
            for h in handles:
                h.wait()
            pltpu.sync_copy(rows_v, out_hbm.at[pl.ds(r0 * _K, _CH * _K)])


def _sc_gather(tab, dsti, srci):
    out = jax.ShapeDtypeStruct((_E_PAD, _DP), jnp.float32)
    f = pl.kernel(
        _gather_body,
        out_type=(out, out),
        mesh=_sc_mesh(),
        compiler_params=pltpu.CompilerParams(use_tc_tiling_on_sc=False),
        scratch_types=[
            pltpu.VMEM((_CH, _K), jnp.int32),
            pltpu.VMEM((_CH * _K, _DP), jnp.float32),
            pltpu.SemaphoreType.DMA,
        ],
    )
    return f(tab, dsti, srci)


def _scatter_body(vals_hbm, idx_hbm, zeros_hbm, part_hbm,
                  acc_sh, idx_v, vals_v, sem):
    from jax.experimental.pallas import tpu_sc as plsc
    c = lax.axis_index("c")
    s = lax.axis_index("s")
    wid = s * _NC + c
    pltpu.sync_copy(zeros_hbm.at[pl.ds(s * _RPT, _RPT)],
                    acc_sh.at[pl.ds(s * _RPT, _RPT)])
    plsc.subcore_barrier()

    @pl.loop(0, _SSTEP)
    def _step(t):
        r0 = wid * _SRT + t * _CH
        pltpu.sync_copy(idx_hbm.at[pl.ds(r0, _CH)], idx_v)
        pltpu.sync_copy(vals_hbm.at[pl.ds(r0 * _K, _CH * _K)], vals_v)
        for j in range(_CH):
            pltpu.sync_copy(vals_v.at[pl.ds(j * _K, _K)],
                            acc_sh.at[idx_v.at[j]], add=True)

    plsc.subcore_barrier()
    pltpu.sync_copy(acc_sh.at[pl.ds(s * _RPT, _RPT)],
                    part_hbm.at[c, pl.ds(s * _RPT, _RPT)])


def _sc_scatter(vals, sidx, zeros16):
    f = pl.kernel(
        _scatter_body,
        out_type=jax.ShapeDtypeStruct((_NC, _N_PAD, _DP), jnp.float32),
        mesh=_sc_mesh(),
        compiler_params=pltpu.CompilerParams(use_tc_tiling_on_sc=False),
        scratch_types=[
            pltpu.VMEM_SHARED((_N_PAD, _DP), jnp.float32),
            pltpu.VMEM((_CH, _K), jnp.int32),
            pltpu.VMEM((_CH * _K, _DP), jnp.float32),
            pltpu.SemaphoreType.DMA,
        ],
    )
    return f(vals, sidx, zeros16)


def _mlp_body(xi_ref, xj_ref, wi_ref, wj_ref, b1_ref, w2_ref, b2_ref,
              w3_ref, b3_ref, o_ref, *, relu_out):
    u = (jnp.dot(xi_ref[...], wi_ref[...], preferred_element_type=jnp.float32)
         + jnp.dot(xj_ref[...], wj_ref[...], preferred_element_type=jnp.float32)
         + b1_ref[...])
    h = jnp.maximum(u, 0.0)
    h = jnp.maximum(
        jnp.dot(h, w2_ref[...], preferred_element_type=jnp.float32)
        + b2_ref[...], 0.0)
    o = jnp.dot(h, w3_ref[...], preferred_element_type=jnp.float32) + b3_ref[...]
    if relu_out:
        o = jnp.maximum(o, 0.0)
    o_ref[...] = o


_MLP_BLK = 4096


def _edge_mlp(xi, xj, wi, wj, b1, w2, b2, w3, b3, relu_out):
    full = lambda r, c: pl.BlockSpec((r, c), lambda i: (0, 0))
    return pl.pallas_call(
        functools.partial(_mlp_body, relu_out=relu_out),
        grid=(_E_PAD // _MLP_BLK,),
        in_specs=[
            pl.BlockSpec((_MLP_BLK, _DP), lambda i: (i, 0)),
            pl.BlockSpec((_MLP_BLK, _DP), lambda i: (i, 0)),
            full(_DP, 32), full(_DP, 32), full(1, 32),
            full(32, 32), full(1, 32),
            full(32, _DP), full(1, _DP),
        ],
        out_specs=pl.BlockSpec((_MLP_BLK, _DP), lambda i: (i, 0)),
        out_shape=jax.ShapeDtypeStruct((_E_PAD, _DP), jnp.float32),
    )(xi, xj, wi, wj, b1.reshape(1, -1), w2, b2.reshape(1, -1),
      w3, b3.reshape(1, -1))


_NB = 6272


def _combine_body(p0_ref, p1_ref, h1_ref, inv_ref):
    s = p0_ref[...] + p1_ref[...]
    inv = 1.0 / jnp.maximum(s[:, 2:3], 1.0)
    cols = lax.broadcasted_iota(jnp.int32, (_NB, _DP), 1)
    h1_ref[...] = jnp.where(cols < 2, s * inv, 0.0)
    inv_ref[...] = inv


def _combine(p0, p1):
    return pl.pallas_call(
        _combine_body,
        grid=(_N_PAD // _NB,),
        in_specs=[pl.BlockSpec((_NB, _DP), lambda i: (i, 0)),
                  pl.BlockSpec((_NB, _DP), lambda i: (i, 0))],
        out_specs=[pl.BlockSpec((_NB, _DP), lambda i: (i, 0)),
                   pl.BlockSpec((_NB, 1), lambda i: (i, 0))],
        out_shape=[jax.ShapeDtypeStruct((_N_PAD, _DP), jnp.float32),
                   jax.ShapeDtypeStruct((_N_PAD, 1), jnp.float32)],
    )(p0, p1)


def _final_body(p0_ref, p1_ref, inv_ref, o_ref):
    s = p0_ref[...] + p1_ref[...]
    o_ref[...] = s[:, 0:4] * inv_ref[...]


def _final(p0, p1, inv):
    return pl.pallas_call(
        _final_body,
        grid=(_N_PAD // _NB,),
        in_specs=[pl.BlockSpec((_NB, _DP), lambda i: (i, 0)),
                  pl.BlockSpec((_NB, _DP), lambda i: (i, 0)),
                  pl.BlockSpec((_NB, 1), lambda i: (i, 0))],
        out_specs=pl.BlockSpec((_NB, 4), lambda i: (i, 0)),
        out_shape=jax.ShapeDtypeStruct((_N_PAD, 4), jnp.float32),
    )(p0, p1, inv)


def kernel(x, edge_index, gamma, beta, eW1, eb1, eW2, eb2, eW3, eb3,
           dW1, db1, dW2, db2, dW3, db3):
    n, d = x.shape

    npad = _E_PAD - _E
    dump = (_N + (jnp.arange(npad, dtype=jnp.int32) % 128))
    dsti = jnp.concatenate([edge_index[1], dump]).reshape(_SR, _K)
    srci = jnp.concatenate([edge_index[0], dump]).reshape(_SR, _K)
    x16 = jnp.zeros((_N_PAD, _DP), jnp.float32).at[:n, :d].set(x)
    zeros16 = jnp.zeros((_N_PAD, _DP), jnp.float32)

    mean = jnp.mean(x, axis=0)
    var = jnp.var(x, axis=0)
    a = gamma * lax.rsqrt(var + 1e-5)
    c = beta - mean * a
    w1t = eW1.T
    w1s = w1t[d:]
    w1d = w1t[:d] - w1s
    w1i = jnp.zeros((_DP, 32), jnp.float32).at[0:d].set(a[:, None] * w1d)
    w1j = jnp.zeros((_DP, 32), jnp.float32).at[0:d].set(a[:, None] * w1s)
    b1c = eb1 + c @ (w1d + w1s)
    w3c = jnp.zeros((32, _DP), jnp.float32).at[:, 0:2].set(eW3.T)
    b3c = jnp.zeros((_DP,), jnp.float32).at[0:2].set(eb3).at[2].set(1.0)

    xi, xj = _sc_gather(x16, dsti, srci)
    m1 = _edge_mlp(xi, xj, w1i, w1j, b1c, eW2.T, eb2, w3c, b3c, True)
    p1 = _sc_scatter(m1, dsti, zeros16)
    h1_16, inv = _combine(p1[0], p1[1])

    hid = 2
    w1t2 = dW1.T
    w2s = w1t2[hid:]
    w2d = w1t2[:hid] - w2s
    w2i = jnp.zeros((_DP, 32), jnp.float32).at[0:hid].set(w2d)
    w2j = jnp.zeros((_DP, 32), jnp.float32).at[0:hid].set(w2s)
    w3c2 = jnp.zeros((32, _DP), jnp.float32).at[:, 0:4].set(dW3.T)
    b3c2 = jnp.zeros((_DP,), jnp.float32).at[0:4].set(db3)

    hi, hj = _sc_gather(h1_16, dsti, srci)
    m2 = _edge_mlp(hi, hj, w2i, w2j, db1, dW2.T, db2, w3c2, b3c2, False)
    p2 = _sc_scatter(m2, dsti, zeros16)
    out = _final(p2[0], p2[1], inv)
    return out[:n]


# speedup vs baseline: 4.6902x; 1.0650x over previous
import functools

import jax
import jax.numpy as jnp
from jax import lax
from jax.experimental import pallas as pl
from jax.experimental.pallas import tpu as pltpu

_N = 100000
_E = 1600000
_NC = 2
_NS = 16
_NW = _NC * _NS
_K = 128
_CH = 8
_DP = 16
_N_PAD = 100352
_E_PAD = 1605632
_SR = _E_PAD // _K
_SRT = _SR // _NW
_SSTEP = _SRT // _CH
_RPT = _N_PAD // _NS


def _sc_mesh():
    from jax.experimental.pallas import tpu_sc as plsc
    return plsc.VectorSubcoreMesh(core_axis_name="c", subcore_axis_name="s")


def _gather_body(tab_hbm, dsti_hbm, srci_hbm, xi_hbm, xj_hbm,
                 idx_v, rows_v, sem):
    c = lax.axis_index("c")
    s = lax.axis_index("s")
    wid = s * _NC + c
    for idx_hbm, out_hbm in ((dsti_hbm, xi_hbm), (srci_hbm, xj_hbm)):
        @pl.loop(0, _SSTEP)
        def _step(t, idx_hbm=idx_hbm, out_hbm=out_hbm):
            r0 = wid * _SRT + t * _CH
            pltpu.sync_copy(idx_hbm.at[pl.ds(r0, _CH)], idx_v)
            handles = [
                pltpu.async_copy(tab_hbm.at[idx_v.at[j]],
                                 rows_v.at[pl.ds(j * _K, _K)], sem)
                for j in range(_CH)
            ]
            for h in handles:
                h.wait()
            pltpu.sync_copy(rows_v, out_hbm.at[pl.ds(r0 * _K, _CH * _K)])


def _sc_gather(tab, dsti, srci):
    out = jax.ShapeDtypeStruct((_E_PAD, _DP), jnp.float32)
    f = pl.kernel(
        _gather_body,
        out_type=(out, out),
        mesh=_sc_mesh(),
        compiler_params=pltpu.CompilerParams(use_tc_tiling_on_sc=False),
        scratch_types=[
            pltpu.VMEM((_CH, _K), jnp.int32),
            pltpu.VMEM((_CH * _K, _DP), jnp.float32),
            pltpu.SemaphoreType.DMA,
        ],
    )
    return f(tab, dsti, srci)


def _scatter_body(vals_hbm, idx_hbm, zeros_hbm, part_hbm,
                  acc_sh, idx_v, vals_v, sem):
    from jax.experimental.pallas import tpu_sc as plsc
    c = lax.axis_index("c")
    s = lax.axis_index("s")
    wid = s * _NC + c
    pltpu.sync_copy(zeros_hbm.at[pl.ds(s * _RPT, _RPT)],
                    acc_sh.at[pl.ds(s * _RPT, _RPT)])
    plsc.subcore_barrier()

    @pl.loop(0, _SSTEP)
    def _step(t):
        r0 = wid * _SRT + t * _CH
        pltpu.sync_copy(idx_hbm.at[pl.ds(r0, _CH)], idx_v)
        pltpu.sync_copy(vals_hbm.at[pl.ds(r0 * _K, _CH * _K)], vals_v)
        for j in range(_CH):
            pltpu.sync_copy(vals_v.at[pl.ds(j * _K, _K)],
                            acc_sh.at[idx_v.at[j]], add=True)

    plsc.subcore_barrier()
    pltpu.sync_copy(acc_sh.at[pl.ds(s * _RPT, _RPT)],
                    part_hbm.at[c, pl.ds(s * _RPT, _RPT)])


def _sc_scatter(vals, sidx, zeros16):
    f = pl.kernel(
        _scatter_body,
        out_type=jax.ShapeDtypeStruct((_NC, _N_PAD, _DP), jnp.float32),
        mesh=_sc_mesh(),
        compiler_params=pltpu.CompilerParams(use_tc_tiling_on_sc=False),
        scratch_types=[
            pltpu.VMEM_SHARED((_N_PAD, _DP), jnp.float32),
            pltpu.VMEM((_CH, _K), jnp.int32),
            pltpu.VMEM((_CH * _K, _DP), jnp.float32),
            pltpu.SemaphoreType.DMA,
        ],
    )
    return f(vals, sidx, zeros16)


def _mlp_body(xi_ref, xj_ref, wi_ref, wj_ref, b1_ref, w2_ref, b2_ref,
              w3_ref, b3_ref, o_ref, *, relu_out):
    xc = jnp.concatenate([xi_ref[...], xj_ref[...]], axis=1)
    w1 = jnp.concatenate([wi_ref[...], wj_ref[...]], axis=0)
    u = (jnp.dot(xc, w1, preferred_element_type=jnp.float32)
         + b1_ref[...])
    h = jnp.maximum(u, 0.0)
    h = jnp.maximum(
        jnp.dot(h, w2_ref[...], preferred_element_type=jnp.float32)
        + b2_ref[...], 0.0)
    o = jnp.dot(h, w3_ref[...], preferred_element_type=jnp.float32) + b3_ref[...]
    if relu_out:
        o = jnp.maximum(o, 0.0)
    o_ref[...] = o


_MLP_BLK = 16384


def _edge_mlp(xi, xj, wi, wj, b1, w2, b2, w3, b3, relu_out):
    full = lambda r, c: pl.BlockSpec((r, c), lambda i: (0, 0))
    return pl.pallas_call(
        functools.partial(_mlp_body, relu_out=relu_out),
        grid=(_E_PAD // _MLP_BLK,),
        in_specs=[
            pl.BlockSpec((_MLP_BLK, _DP), lambda i: (i, 0)),
            pl.BlockSpec((_MLP_BLK, _DP), lambda i: (i, 0)),
            full(_DP, 32), full(_DP, 32), full(1, 32),
            full(32, 32), full(1, 32),
            full(32, _DP), full(1, _DP),
        ],
        out_specs=pl.BlockSpec((_MLP_BLK, _DP), lambda i: (i, 0)),
        out_shape=jax.ShapeDtypeStruct((_E_PAD, _DP), jnp.float32),
    )(xi, xj, wi, wj, b1.reshape(1, -1), w2, b2.reshape(1, -1),
      w3, b3.reshape(1, -1))


_NB = 6272


def _combine_body(p0_ref, p1_ref, h1_ref, inv_ref):
    s = p0_ref[...] + p1_ref[...]
    inv = 1.0 / jnp.maximum(s[:, 2:3], 1.0)
    cols = lax.broadcasted_iota(jnp.int32, (_NB, _DP), 1)
    h1_ref[...] = jnp.where(cols < 2, s * inv, 0.0)
    inv_ref[...] = inv


def _combine(p0, p1):
    return pl.pallas_call(
        _combine_body,
        grid=(_N_PAD // _NB,),
        in_specs=[pl.BlockSpec((_NB, _DP), lambda i: (i, 0)),
                  pl.BlockSpec((_NB, _DP), lambda i: (i, 0))],
        out_specs=[pl.BlockSpec((_NB, _DP), lambda i: (i, 0)),
                   pl.BlockSpec((_NB, 1), lambda i: (i, 0))],
        out_shape=[jax.ShapeDtypeStruct((_N_PAD, _DP), jnp.float32),
                   jax.ShapeDtypeStruct((_N_PAD, 1), jnp.float32)],
    )(p0, p1)


def _final_body(p0_ref, p1_ref, inv_ref, o_ref):
    s = p0_ref[...] + p1_ref[...]
    o_ref[...] = s[:, 0:4] * inv_ref[...]


def _final(p0, p1, inv):
    return pl.pallas_call(
        _final_body,
        grid=(_N_PAD // _NB,),
        in_specs=[pl.BlockSpec((_NB, _DP), lambda i: (i, 0)),
                  pl.BlockSpec((_NB, _DP), lambda i: (i, 0)),
                  pl.BlockSpec((_NB, 1), lambda i: (i, 0))],
        out_specs=pl.BlockSpec((_NB, 4), lambda i: (i, 0)),
        out_shape=jax.ShapeDtypeStruct((_N_PAD, 4), jnp.float32),
    )(p0, p1, inv)


def kernel(x, edge_index, gamma, beta, eW1, eb1, eW2, eb2, eW3, eb3,
           dW1, db1, dW2, db2, dW3, db3):
    n, d = x.shape

    npad = _E_PAD - _E
    dump = (_N + (jnp.arange(npad, dtype=jnp.int32) % 128))
    dsti = jnp.concatenate([edge_index[1], dump]).reshape(_SR, _K)
    srci = jnp.concatenate([edge_index[0], dump]).reshape(_SR, _K)
    x16 = jnp.zeros((_N_PAD, _DP), jnp.float32).at[:n, :d].set(x)
    zeros16 = jnp.zeros((_N_PAD, _DP), jnp.float32)

    mean = jnp.mean(x, axis=0)
    var = jnp.var(x, axis=0)
    a = gamma * lax.rsqrt(var + 1e-5)
    c = beta - mean * a
    w1t = eW1.T
    w1s = w1t[d:]
    w1d = w1t[:d] - w1s
    w1i = jnp.zeros((_DP, 32), jnp.float32).at[0:d].set(a[:, None] * w1d)
    w1j = jnp.zeros((_DP, 32), jnp.float32).at[0:d].set(a[:, None] * w1s)
    b1c = eb1 + c @ (w1d + w1s)
    w3c = jnp.zeros((32, _DP), jnp.float32).at[:, 0:2].set(eW3.T)
    b3c = jnp.zeros((_DP,), jnp.float32).at[0:2].set(eb3).at[2].set(1.0)

    xi, xj = _sc_gather(x16, dsti, srci)
    m1 = _edge_mlp(xi, xj, w1i, w1j, b1c, eW2.T, eb2, w3c, b3c, True)
    p1 = _sc_scatter(m1, dsti, zeros16)
    h1_16, inv = _combine(p1[0], p1[1])

    hid = 2
    w1t2 = dW1.T
    w2s = w1t2[hid:]
    w2d = w1t2[:hid] - w2s
    w2i = jnp.zeros((_DP, 32), jnp.float32).at[0:hid].set(w2d)
    w2j = jnp.zeros((_DP, 32), jnp.float32).at[0:hid].set(w2s)
    w3c2 = jnp.zeros((32, _DP), jnp.float32).at[:, 0:4].set(dW3.T)
    b3c2 = jnp.zeros((_DP,), jnp.float32).at[0:4].set(db3)

    hi, hj = _sc_gather(h1_16, dsti, srci)
    m2 = _edge_mlp(hi, hj, w2i, w2j, db1, dW2.T, db2, w3c2, b3c2, False)
    p2 = _sc_scatter(m2, dsti, zeros16)
    out = _final(p2[0], p2[1], inv)
    return out[:n]
